# baseline (speedup 1.0000x reference)
import functools

import jax
import jax.numpy as jnp
from jax import lax
from jax.experimental import pallas as pl
from jax.experimental.pallas import tpu as pltpu
from jax.experimental.pallas import tpu_sc as plsc

NC = 2
NS = 16
L = 16

N = 10000
E = 320000
C = 128
CHUNKS = E // L
CPW = CHUNKS // (NC * NS)
NPAD = 10240
ROWS_PER_TILE = NPAD // NS
DR = NPAD // L
DR_PER_TILE = DR // NS

BN = 1024


def _proj_body(x_ref, wl_ref, bl_ref, wr_ref, br_ref, xl_ref, xr_ref):
    xb = x_ref[...]
    dn = (((1,), (1,)), ((), ()))
    xl_ref[...] = lax.dot_general(xb, wl_ref[...], dn,
                                  preferred_element_type=jnp.float32) + bl_ref[...]
    xr_ref[...] = lax.dot_general(xb, wr_ref[...], dn,
                                  preferred_element_type=jnp.float32) + br_ref[...]


def _proj(x, Wl, bl, Wr, br):
    grid = 10
    return pl.pallas_call(
        _proj_body,
        grid=(grid,),
        in_specs=[
            pl.BlockSpec((N // 10, C), lambda i: (i, 0)),
            pl.BlockSpec((C, C), lambda i: (0, 0)),
            pl.BlockSpec((1, C), lambda i: (0, 0)),
            pl.BlockSpec((C, C), lambda i: (0, 0)),
            pl.BlockSpec((1, C), lambda i: (0, 0)),
        ],
        out_specs=[
            pl.BlockSpec((N // 10, C), lambda i: (i, 0)),
            pl.BlockSpec((N // 10, C), lambda i: (i, 0)),
        ],
        out_shape=[
            jax.ShapeDtypeStruct((N, C), jnp.float32),
            jax.ShapeDtypeStruct((N, C), jnp.float32),
        ],
    )(x, Wl, bl.reshape(1, C), Wr, br.reshape(1, C))


def _sc_edge_body(src_hbm, dst_hbm, xl_hbm, xr_hbm, zeros_hbm, zeros1_hbm,
                  att_hbm, num_hbm, den_hbm, src_v, dst_v, u0, v0, u1, v1,
                  val0, val1, acc_m, att_v, den_l, num_sh,
                  su0, sv0, su1, sv1, sc0, sc1):
    c = lax.axis_index("c")
    s = lax.axis_index("s")
    w = c * NS + s
    r0 = pl.multiple_of(s * ROWS_PER_TILE, 8)

    pltpu.sync_copy(zeros_hbm.at[pl.ds(r0, ROWS_PER_TILE)],
                    num_sh.at[pl.ds(r0, ROWS_PER_TILE)])
    pltpu.sync_copy(zeros1_hbm, den_l)
    e0 = pl.multiple_of(w * (CPW * L), 8)
    pltpu.sync_copy(src_hbm.at[pl.ds(e0, CPW * L)], src_v)
    pltpu.sync_copy(dst_hbm.at[pl.ds(e0, CPW * L)], dst_v)
    pltpu.sync_copy(att_hbm, att_v)
    plsc.subcore_barrier()

    lanes = lax.iota(jnp.int32, L)
    att_s = [att_v[pl.ds(j * L, L)] for j in range(C // L)]

    def ids(g):
        go = pl.multiple_of(g * L, 8)
        return src_v[pl.ds(go, L)], dst_v[pl.ds(go, L)]

    def issue(g, ub, vb, su, sv):
        sid, did = ids(g)
        pltpu.async_copy(xl_hbm.at[sid], ub, su)
        pltpu.async_copy(xr_hbm.at[did], vb, sv)

    def compute(g, ub, vb, su, sv, valb, scsem):
        sid, did = ids(g)
        pltpu.make_async_copy(xl_hbm.at[sid], ub, su).wait()
        pltpu.make_async_copy(xr_hbm.at[did], vb, sv).wait()

        for e in range(L):
            z0 = ub[e, pl.ds(0, L)] + vb[e, pl.ds(0, L)]
            acc = att_s[0] * jnp.maximum(z0, 0.2 * z0)
            for j in range(1, C // L):
                z = ub[e, pl.ds(j * L, L)] + vb[e, pl.ds(j * L, L)]
                acc = acc + att_s[j] * jnp.maximum(z, 0.2 * z)
            acc_m[e, pl.ds(0, L)] = acc
        alpha = plsc.load_gather(acc_m, [lanes, jnp.full((L,), 0, jnp.int32)])
        for l in range(1, L):
            alpha = alpha + plsc.load_gather(
                acc_m, [lanes, jnp.full((L,), l, jnp.int32)])
        ex = jnp.exp(alpha)

        for e in range(L):
            ex_e = ex[e]
            for j in range(C // L):
                valb[e, pl.ds(j * L, L)] = ex_e * ub[e, pl.ds(j * L, L)]
            off = pl.multiple_of(did[e] & -L, 8)
            row = den_l[pl.ds(off, L)]
            den_l[pl.ds(off, L)] = row + jnp.where(
                lanes == (did[e] & (L - 1)), ex_e, 0.0)
        pltpu.sync_copy(valb, num_sh.at[did], add=True)

    def wait_scatter(valb, did, scsem):
        pltpu.make_async_copy(valb, num_sh.at[did], scsem).wait()

    issue(0, u0, v0, su0, sv0)

    def pair_body(h, carry):
        g = h * 2
        _, did0 = ids(g)
        _, did1 = ids(g + 1)
        issue(g + 1, u1, v1, su1, sv1)

        compute(g, u0, v0, su0, sv0, val0, sc0)
        issue(g + 2, u0, v0, su0, sv0)
        compute(g + 1, u1, v1, su1, sv1, val1, sc1)
        return carry

    lax.fori_loop(0, CPW // 2, pair_body, 0)
    compute(CPW - 1, u0, v0, su0, sv0, val0, sc0)

    plsc.subcore_barrier()
    pltpu.sync_copy(num_sh.at[pl.ds(r0, ROWS_PER_TILE)],
                    num_hbm.at[c, pl.ds(r0, ROWS_PER_TILE)])
    do = pl.multiple_of(w * (DR * L), 8)
    pltpu.sync_copy(den_l, den_hbm.at[pl.ds(do, DR * L)])


_sc_edge = functools.partial(
    pl.kernel,
    out_type=(
        jax.ShapeDtypeStruct((NC, NPAD, C), jnp.float32),
        jax.ShapeDtypeStruct((NC * NS * DR * L,), jnp.float32),
    ),
    mesh=plsc.VectorSubcoreMesh(core_axis_name="c", subcore_axis_name="s",
                                num_cores=NC, num_subcores=NS),
    compiler_params=pltpu.CompilerParams(needs_layout_passes=False),
    scratch_types=[
        pltpu.VMEM((CPW * L,), jnp.int32),
        pltpu.VMEM((CPW * L,), jnp.int32),
        pltpu.VMEM((L, C), jnp.float32),
        pltpu.VMEM((L, C), jnp.float32),
        pltpu.VMEM((L, C), jnp.float32),
        pltpu.VMEM((L, C), jnp.float32),
        pltpu.VMEM((L, C), jnp.float32),
        pltpu.VMEM((L, C), jnp.float32),
        pltpu.VMEM((L, L), jnp.float32),
        pltpu.VMEM((C,), jnp.float32),
        pltpu.VMEM((DR * L,), jnp.float32),
        pltpu.VMEM_SHARED((NPAD, C), jnp.float32),
        pltpu.SemaphoreType.DMA,
        pltpu.SemaphoreType.DMA,
        pltpu.SemaphoreType.DMA,
        pltpu.SemaphoreType.DMA,
        pltpu.SemaphoreType.DMA,
        pltpu.SemaphoreType.DMA,
    ],
)(_sc_edge_body)


def _combine_body(num_ref, den_ref, bias_ref, o_ref):
    num = num_ref[0] + num_ref[1]
    den = jnp.sum(den_ref[...], axis=1, keepdims=True)
    o_ref[...] = num / (den + 1e-16) + bias_ref[...]


def _combine(num_p, den_t, bias):
    grid = NPAD // BN
    return pl.pallas_call(
        _combine_body,
        grid=(grid,),
        in_specs=[
            pl.BlockSpec((NC, BN, C), lambda i: (0, i, 0)),
            pl.BlockSpec((BN, NC * NS), lambda i: (i, 0)),
            pl.BlockSpec((1, C), lambda i: (0, 0)),
        ],
        out_specs=pl.BlockSpec((BN, C), lambda i: (i, 0)),
        out_shape=jax.ShapeDtypeStruct((NPAD, C), jnp.float32),
    )(num_p, den_t, bias.reshape(1, C))


def kernel(x, edge_index, Wl, bl, Wr, br, att, bias, Wg, bg):
    src = edge_index[0]
    dst = edge_index[1]
    xl, xr = _proj(x, Wl, bl, Wr, br)
    zeros = jnp.zeros((NPAD, C), jnp.float32)
    zeros1 = jnp.zeros((DR * L,), jnp.float32)
    num_p, den_p = _sc_edge(src, dst, xl, xr, zeros, zeros1, att)
    den_t = den_p.reshape(NC * NS, NPAD).T
    return _combine(num_p, den_t, bias)[:N]


# speedup vs baseline: 1.6009x; 1.6009x over previous
import functools

import jax
import jax.numpy as jnp
from jax import lax
from jax.experimental import pallas as pl
from jax.experimental.pallas import tpu as pltpu
from jax.experimental.pallas import tpu_sc as plsc

NC = 2
NS = 16
L = 16

N = 10000
E = 320000
C = 128
CHUNKS = E // L
CPW = CHUNKS // (NC * NS)
NPAD = 10240
ROWS_PER_TILE = NPAD // NS
DR = NPAD // L
DR_PER_TILE = DR // NS

BN = 1024


def _proj_body(x_ref, wl_ref, bl_ref, wr_ref, br_ref, xl_ref, xr_ref):
    xb = x_ref[...]
    dn = (((1,), (1,)), ((), ()))
    xl_ref[...] = lax.dot_general(xb, wl_ref[...], dn,
                                  preferred_element_type=jnp.float32) + bl_ref[...]
    xr_ref[...] = lax.dot_general(xb, wr_ref[...], dn,
                                  preferred_element_type=jnp.float32) + br_ref[...]


def _proj(x, Wl, bl, Wr, br):
    grid = 10
    return pl.pallas_call(
        _proj_body,
        grid=(grid,),
        in_specs=[
            pl.BlockSpec((N // 10, C), lambda i: (i, 0)),
            pl.BlockSpec((C, C), lambda i: (0, 0)),
            pl.BlockSpec((1, C), lambda i: (0, 0)),
            pl.BlockSpec((C, C), lambda i: (0, 0)),
            pl.BlockSpec((1, C), lambda i: (0, 0)),
        ],
        out_specs=[
            pl.BlockSpec((N // 10, C), lambda i: (i, 0)),
            pl.BlockSpec((N // 10, C), lambda i: (i, 0)),
        ],
        out_shape=[
            jax.ShapeDtypeStruct((N, C), jnp.float32),
            jax.ShapeDtypeStruct((N, C), jnp.float32),
        ],
    )(x, Wl, bl.reshape(1, C), Wr, br.reshape(1, C))


def _sc_edge_body(pk_hbm, xl_hbm, xr_hbm, zeros_hbm, att_hbm,
                  num_hbm, den_hbm, pk_v,
                  u0, v0, u1, v1, u2, v2,
                  val0, val1, val2, dval0, dval1, dval2,
                  acc_m, att_v, num_sh, den_sh,
                  su0, sv0, su1, sv1, su2, sv2, scn, scd):
    c = lax.axis_index("c")
    s = lax.axis_index("s")
    w = c * NS + s
    r0 = pl.multiple_of(s * ROWS_PER_TILE, 8)
    d0 = pl.multiple_of(s * DR_PER_TILE, 8)

    pltpu.sync_copy(zeros_hbm.at[pl.ds(r0, ROWS_PER_TILE)],
                    num_sh.at[pl.ds(r0, ROWS_PER_TILE)])
    pltpu.sync_copy(zeros_hbm.at[pl.ds(d0, DR_PER_TILE)],
                    den_sh.at[pl.ds(d0, DR_PER_TILE)])
    e0 = pl.multiple_of(w * (CPW * L), 8)
    pltpu.sync_copy(pk_hbm.at[pl.ds(e0, CPW * L)], pk_v)
    pltpu.sync_copy(att_hbm, att_v)
    zero_l = jnp.zeros((L,), jnp.float32)
    for dv in (dval0, dval1, dval2):
        for e in range(L):
            for j in range(C // L):
                dv[e, pl.ds(j * L, L)] = zero_l
    plsc.subcore_barrier()

    lanes = lax.iota(jnp.int32, L)
    att_s = [att_v[pl.ds(j * L, L)] for j in range(C // L)]

    def ids(g):
        go = pl.multiple_of(g * L, 8)
        pk = pk_v[pl.ds(go, L)]
        return pk & (16384 - 1), lax.shift_right_logical(pk, 14)

    def issue(g, ub, vb, su, sv):
        sid, did = ids(g)
        pltpu.async_copy(xl_hbm.at[sid], ub, su)
        pltpu.async_copy(xr_hbm.at[did], vb, sv)

    def drain(valb, dvalb, did):
        pltpu.make_async_copy(valb, num_sh.at[did], scn).wait()
        pltpu.make_async_copy(
            dvalb, den_sh.at[lax.shift_right_logical(did, 4)], scd).wait()

    def compute(g, ub, vb, su, sv, valb, dvalb, prev, prev_cond):
        sid, did = ids(g)
        pltpu.make_async_copy(xl_hbm.at[sid], ub, su).wait()
        pltpu.make_async_copy(xr_hbm.at[did], vb, sv).wait()

        for e in range(L):
            z0 = ub[e, pl.ds(0, L)] + vb[e, pl.ds(0, L)]
            acc = att_s[0] * jnp.maximum(z0, 0.2 * z0)
            for j in range(1, C // L):
                z = ub[e, pl.ds(j * L, L)] + vb[e, pl.ds(j * L, L)]
                acc = acc + att_s[j] * jnp.maximum(z, 0.2 * z)
            acc_m[e, pl.ds(0, L)] = acc
        alpha = plsc.load_gather(acc_m, [lanes, jnp.full((L,), 0, jnp.int32)])
        for l in range(1, L):
            alpha = alpha + plsc.load_gather(
                acc_m, [lanes, jnp.full((L,), l, jnp.int32)])
        ex = jnp.exp(alpha)

        for e in range(L):
            ex_e = ex[e]
            for j in range(C // L):
                valb[e, pl.ds(j * L, L)] = ex_e * ub[e, pl.ds(j * L, L)]
            dvalb[e, pl.ds(0, L)] = jnp.where(
                lanes == (did[e] & (L - 1)), ex_e, 0.0)

        if prev_cond is None:
            drain(prev[0], prev[1], did)
        else:
            @pl.when(prev_cond)
            def _():
                drain(prev[0], prev[1], did)
        pltpu.async_copy(valb, num_sh.at[did], scn, add=True)
        pltpu.async_copy(dvalb, den_sh.at[lax.shift_right_logical(did, 4)],
                         scd, add=True)

    issue(0, u0, v0, su0, sv0)
    issue(1, u1, v1, su1, sv1)
    issue(2, u2, v2, su2, sv2)

    def triple_body(h, carry):
        g = h * 3
        compute(g, u0, v0, su0, sv0, val0, dval0, (val2, dval2), h > 0)
        issue(g + 3, u0, v0, su0, sv0)
        compute(g + 1, u1, v1, su1, sv1, val1, dval1, (val0, dval0), None)

        @pl.when(g + 4 < CPW)
        def _():
            issue(g + 4, u1, v1, su1, sv1)
        compute(g + 2, u2, v2, su2, sv2, val2, dval2, (val1, dval1), None)

        @pl.when(g + 5 < CPW)
        def _():
            issue(g + 5, u2, v2, su2, sv2)
        return carry

    lax.fori_loop(0, CPW // 3, triple_body, 0)
    compute(CPW - 1, u0, v0, su0, sv0, val0, dval0, (val2, dval2), None)
    _, did_last = ids(CPW - 1)
    drain(val0, dval0, did_last)

    plsc.subcore_barrier()
    pltpu.sync_copy(num_sh.at[pl.ds(r0, ROWS_PER_TILE)],
                    num_hbm.at[c, pl.ds(r0, ROWS_PER_TILE)])
    pltpu.sync_copy(den_sh.at[pl.ds(d0, DR_PER_TILE)],
                    den_hbm.at[c, pl.ds(d0, DR_PER_TILE)])


_sc_edge = functools.partial(
    pl.kernel,
    out_type=(
        jax.ShapeDtypeStruct((NC, NPAD, C), jnp.float32),
        jax.ShapeDtypeStruct((NC, DR, C), jnp.float32),
    ),
    mesh=plsc.VectorSubcoreMesh(core_axis_name="c", subcore_axis_name="s",
                                num_cores=NC, num_subcores=NS),
    compiler_params=pltpu.CompilerParams(needs_layout_passes=False),
    scratch_types=[
        pltpu.VMEM((CPW * L,), jnp.int32),
        pltpu.VMEM((L, C), jnp.float32),
        pltpu.VMEM((L, C), jnp.float32),
        pltpu.VMEM((L, C), jnp.float32),
        pltpu.VMEM((L, C), jnp.float32),
        pltpu.VMEM((L, C), jnp.float32),
        pltpu.VMEM((L, C), jnp.float32),
        pltpu.VMEM((L, C), jnp.float32),
        pltpu.VMEM((L, C), jnp.float32),
        pltpu.VMEM((L, C), jnp.float32),
        pltpu.VMEM((L, C), jnp.float32),
        pltpu.VMEM((L, C), jnp.float32),
        pltpu.VMEM((L, C), jnp.float32),
        pltpu.VMEM((L, L), jnp.float32),
        pltpu.VMEM((C,), jnp.float32),
        pltpu.VMEM_SHARED((NPAD, C), jnp.float32),
        pltpu.VMEM_SHARED((DR, C), jnp.float32),
        pltpu.SemaphoreType.DMA,
        pltpu.SemaphoreType.DMA,
        pltpu.SemaphoreType.DMA,
        pltpu.SemaphoreType.DMA,
        pltpu.SemaphoreType.DMA,
        pltpu.SemaphoreType.DMA,
        pltpu.SemaphoreType.DMA,
        pltpu.SemaphoreType.DMA,
    ],
)(_sc_edge_body)


def _combine_body(num_ref, den_ref, bias_ref, o_ref):
    num = num_ref[0] + num_ref[1]
    den = jnp.sum(den_ref[...], axis=1, keepdims=True)
    o_ref[...] = num / (den + 1e-16) + bias_ref[...]


def _combine(num_p, den_t, bias):
    grid = NPAD // BN
    return pl.pallas_call(
        _combine_body,
        grid=(grid,),
        in_specs=[
            pl.BlockSpec((NC, BN, C), lambda i: (0, i, 0)),
            pl.BlockSpec((BN, NC), lambda i: (i, 0)),
            pl.BlockSpec((1, C), lambda i: (0, 0)),
        ],
        out_specs=pl.BlockSpec((BN, C), lambda i: (i, 0)),
        out_shape=jax.ShapeDtypeStruct((NPAD, C), jnp.float32),
    )(num_p, den_t, bias.reshape(1, C))


def kernel(x, edge_index, Wl, bl, Wr, br, att, bias, Wg, bg):
    packed = edge_index[0] + edge_index[1] * 16384
    xl, xr = _proj(x, Wl, bl, Wr, br)
    zeros = jnp.zeros((NPAD, C), jnp.float32)
    num_p, den_p = _sc_edge(packed, xl, xr, zeros, att)
    den_t = den_p[:, :, :L].reshape(NC, NPAD).T
    return _combine(num_p, den_t, bias)[:N]
